# baseline (speedup 1.0000x reference)
import functools

import jax
import jax.numpy as jnp
from jax import lax
from jax.experimental import pallas as pl
from jax.experimental.pallas import tpu as pltpu
from jax.experimental.pallas import tpu_sc as plsc

N_SP = 1024
KNN = 8
ROWS = 1024
NW = 32
CHUNK = 1568
NPAD = NW * CHUNK
NGRP = CHUNK // 16


def _table_body(spt_ref, sdr_ref, sdt_ref, rad_ref, tbl_ref):
    sx = spt_ref[0:1, :]
    sy = spt_ref[1:2, :]
    sz = spt_ref[2:3, :]
    wx = sdr_ref[0:1, :]
    wy = sdr_ref[1:2, :]
    wz = sdr_ref[2:3, :]
    tx = sdt_ref[0:1, :]
    ty = sdt_ref[1:2, :]
    tz = sdt_ref[2:3, :]
    rad = rad_ref[...]

    th2 = wx * wx + wy * wy + wz * wz
    th = jnp.sqrt(th2)
    small = th < 1e-6
    safe = jnp.where(small, 1.0, th)
    inv = 1.0 / safe
    kx = wx * inv
    ky = wy * inv
    kz = wz * inv
    ct = jnp.cos(th)
    st = jnp.sin(th)
    omc = 1.0 - ct
    a00 = jnp.where(small, 1.0, ct + omc * kx * kx)
    a01 = jnp.where(small, -wz, omc * kx * ky - st * kz)
    a02 = jnp.where(small, wy, omc * kx * kz + st * ky)
    a10 = jnp.where(small, wz, omc * ky * kx + st * kz)
    a11 = jnp.where(small, 1.0, ct + omc * ky * ky)
    a12 = jnp.where(small, -wx, omc * ky * kz - st * kx)
    a20 = jnp.where(small, -wy, omc * kz * kx - st * ky)
    a21 = jnp.where(small, wx, omc * kz * ky + st * kx)
    a22 = jnp.where(small, 1.0, ct + omc * kz * kz)
    bx = sx - (a00 * sx + a01 * sy + a02 * sz) + tx
    by = sy - (a10 * sx + a11 * sy + a12 * sz) + ty
    bz = sz - (a20 * sx + a21 * sy + a22 * sz) + tz
    s2 = sx * sx + sy * sy + sz * sz
    rinv2 = 1.0 / (rad * rad)
    zero = jnp.zeros_like(sx)
    rows = [a00, a01, a02, a10, a11, a12, a20, a21, a22,
            bx, by, bz, wx, wy, wz, zero, s2, rinv2,
            zero, zero, zero, zero, zero, zero]
    for i, v in enumerate(rows):
        tbl_ref[i:i + 1, :] = v


def _main_body(pts_ref, sp_ref, aux_ref, g_ref, nb_ref):
    Pt = pts_ref[...]
    sp = sp_ref[...]
    s2 = aux_ref[:, 0:1]
    rinv2 = aux_ref[:, 1:2]

    p2 = jnp.sum(Pt * Pt, axis=0, keepdims=True)
    sps = jax.lax.dot_general(sp, Pt, (((1,), (0,)), ((), ())),
                              preferred_element_type=jnp.float32)
    d2 = jnp.maximum(s2 + p2 - 2.0 * sps, 0.0)
    q = d2 * rinv2 + (1e-12 * rinv2)

    iota = jax.lax.broadcasted_iota(jnp.int32, q.shape, 0).astype(jnp.float32)
    BIG = jnp.float32(3.0e38)
    FBIG = jnp.float32(1e9)
    es = []
    q0 = None
    Z = None
    for k in range(KNN):
        m = jnp.min(q, axis=0, keepdims=True)
        eq = q == m
        idxf = jnp.min(jnp.where(eq, iota, FBIG), axis=0,
                       keepdims=True)
        if k == 0:
            q0 = m
            e = jnp.ones_like(m)
            Z = e
        else:
            e = jnp.exp(2.0 * (q0 - m))
            Z = Z + e
        q = jnp.where(iota == idxf, BIG, q)
        nb_ref[k:k + 1, :] = idxf.astype(jnp.int32)
        es.append(e)
    zinv = 1.0 / Z
    for k in range(KNN):
        g_ref[k:k + 1, :] = es[k] * zinv


def _sc_combine(pts_hbm, nb_hbm, g_hbm, tbl_hbm, oxyz_hbm, odr_hbm,
                pts_v, nb_v, g_v, tbl_v, ox_v, or_v):
    c = lax.axis_index("c")
    s = lax.axis_index("s")
    wid = s * 2 + c
    base = wid * CHUNK
    for p in range(3):
        pltpu.sync_copy(pts_hbm.at[pl.ds(p * NPAD + base, CHUNK)],
                        pts_v.at[pl.ds(p * CHUNK, CHUNK)])
    for k in range(KNN):
        pltpu.sync_copy(nb_hbm.at[pl.ds(k * NPAD + base, CHUNK)],
                        nb_v.at[pl.ds(k * CHUNK, CHUNK)])
        pltpu.sync_copy(g_hbm.at[pl.ds(k * NPAD + base, CHUNK)],
                        g_v.at[pl.ds(k * CHUNK, CHUNK)])
    pltpu.sync_copy(tbl_hbm, tbl_v)

    def group(gi, carry):
        b = gi * 16
        accs = [jnp.zeros((16,), jnp.float32) for _ in range(15)]
        for k in range(KNN):
            nk = nb_v[pl.ds(k * CHUNK + b, 16)]
            gk = g_v[pl.ds(k * CHUNK + b, 16)]
            row = nk * 16
            for j in range(15):
                accs[j] = accs[j] + gk * plsc.load_gather(tbl_v, [row + j])
        px = pts_v[pl.ds(b, 16)]
        py = pts_v[pl.ds(CHUNK + b, 16)]
        pz = pts_v[pl.ds(2 * CHUNK + b, 16)]
        ox_v[pl.ds(b, 16)] = accs[0] * px + accs[1] * py + accs[2] * pz + accs[9]
        ox_v[pl.ds(CHUNK + b, 16)] = accs[3] * px + accs[4] * py + accs[5] * pz + accs[10]
        ox_v[pl.ds(2 * CHUNK + b, 16)] = accs[6] * px + accs[7] * py + accs[8] * pz + accs[11]
        or_v[pl.ds(b, 16)] = accs[12]
        or_v[pl.ds(CHUNK + b, 16)] = accs[13]
        or_v[pl.ds(2 * CHUNK + b, 16)] = accs[14]
        return carry

    lax.fori_loop(0, NGRP, group, 0)

    for p in range(3):
        pltpu.sync_copy(ox_v.at[pl.ds(p * CHUNK, CHUNK)],
                        oxyz_hbm.at[pl.ds(p * NPAD + base, CHUNK)])
        pltpu.sync_copy(or_v.at[pl.ds(p * CHUNK, CHUNK)],
                        odr_hbm.at[pl.ds(p * NPAD + base, CHUNK)])


_sc_combine_call = functools.partial(
    pl.kernel,
    mesh=plsc.VectorSubcoreMesh(core_axis_name="c", subcore_axis_name="s"),
    compiler_params=pltpu.CompilerParams(needs_layout_passes=False),
    out_type=[
        jax.ShapeDtypeStruct((3 * NPAD,), jnp.float32),
        jax.ShapeDtypeStruct((3 * NPAD,), jnp.float32),
    ],
    scratch_types=[
        pltpu.VMEM((3 * CHUNK,), jnp.float32),
        pltpu.VMEM((KNN * CHUNK,), jnp.int32),
        pltpu.VMEM((KNN * CHUNK,), jnp.float32),
        pltpu.VMEM((N_SP * 16,), jnp.float32),
        pltpu.VMEM((3 * CHUNK,), jnp.float32),
        pltpu.VMEM((3 * CHUNK,), jnp.float32),
    ],
)(_sc_combine)


@jax.jit
def kernel(points, sp_delta_t, sp_delta_r, sp_xyz, radius):
    n = points.shape[0]
    spt = sp_xyz.T
    sdr = sp_delta_r.T
    sdt = sp_delta_t.T
    rad2 = radius.reshape(1, N_SP)

    tbl = pl.pallas_call(
        _table_body,
        out_shape=jax.ShapeDtypeStruct((24, N_SP), jnp.float32),
    )(spt, sdr, sdt, rad2)

    pts_t = points.T
    aux = tbl[16:18, :].T

    nblocks = (n + ROWS - 1) // ROWS
    g_t, nb_t = pl.pallas_call(
        _main_body,
        grid=(nblocks,),
        in_specs=[
            pl.BlockSpec((3, ROWS), lambda i: (0, i)),
            pl.BlockSpec((N_SP, 3), lambda i: (0, 0)),
            pl.BlockSpec((N_SP, 2), lambda i: (0, 0)),
        ],
        out_specs=[
            pl.BlockSpec((KNN, ROWS), lambda i: (0, i)),
            pl.BlockSpec((KNN, ROWS), lambda i: (0, i)),
        ],
        out_shape=[
            jax.ShapeDtypeStruct((KNN, n), jnp.float32),
            jax.ShapeDtypeStruct((KNN, n), jnp.int32),
        ],
        compiler_params=pltpu.CompilerParams(
            dimension_semantics=("arbitrary",),
        ),
    )(pts_t, sp_xyz, aux)

    pad = NPAD - n
    pts_flat = jnp.pad(pts_t, ((0, 0), (0, pad))).reshape(-1)
    nb_flat = jnp.pad(nb_t, ((0, 0), (0, pad))).reshape(-1)
    g_flat = jnp.pad(g_t, ((0, 0), (0, pad))).reshape(-1)
    tbl_rows = tbl[0:16, :].T.reshape(-1)

    oxyz_t, odr_t = _sc_combine_call(pts_flat, nb_flat, g_flat, tbl_rows)
    dxyz = oxyz_t.reshape(3, NPAD)[:, :n].T
    dr = odr_t.reshape(3, NPAD)[:, :n].T
    return (dxyz, dr, g_t.T, nb_t.T)


# speedup vs baseline: 1.1272x; 1.1272x over previous
import functools

import jax
import jax.numpy as jnp
from jax import lax
from jax.experimental import pallas as pl
from jax.experimental.pallas import tpu as pltpu
from jax.experimental.pallas import tpu_sc as plsc

N_SP = 1024
KNN = 8
ROWS = 1024
NW = 32
CHUNK = 1568
NPAD = NW * CHUNK
NGRP = CHUNK // 16


def _table_body(spt_ref, sdr_ref, sdt_ref, rad_ref, tbl_ref):
    sx = spt_ref[0:1, :]
    sy = spt_ref[1:2, :]
    sz = spt_ref[2:3, :]
    wx = sdr_ref[0:1, :]
    wy = sdr_ref[1:2, :]
    wz = sdr_ref[2:3, :]
    tx = sdt_ref[0:1, :]
    ty = sdt_ref[1:2, :]
    tz = sdt_ref[2:3, :]
    rad = rad_ref[...]

    th2 = wx * wx + wy * wy + wz * wz
    th = jnp.sqrt(th2)
    small = th < 1e-6
    safe = jnp.where(small, 1.0, th)
    inv = 1.0 / safe
    kx = wx * inv
    ky = wy * inv
    kz = wz * inv
    ct = jnp.cos(th)
    st = jnp.sin(th)
    omc = 1.0 - ct
    a00 = jnp.where(small, 1.0, ct + omc * kx * kx)
    a01 = jnp.where(small, -wz, omc * kx * ky - st * kz)
    a02 = jnp.where(small, wy, omc * kx * kz + st * ky)
    a10 = jnp.where(small, wz, omc * ky * kx + st * kz)
    a11 = jnp.where(small, 1.0, ct + omc * ky * ky)
    a12 = jnp.where(small, -wx, omc * ky * kz - st * kx)
    a20 = jnp.where(small, -wy, omc * kz * kx - st * ky)
    a21 = jnp.where(small, wx, omc * kz * ky + st * kx)
    a22 = jnp.where(small, 1.0, ct + omc * kz * kz)
    bx = sx - (a00 * sx + a01 * sy + a02 * sz) + tx
    by = sy - (a10 * sx + a11 * sy + a12 * sz) + ty
    bz = sz - (a20 * sx + a21 * sy + a22 * sz) + tz
    s2 = sx * sx + sy * sy + sz * sz
    rinv2 = 1.0 / (rad * rad)
    zero = jnp.zeros_like(sx)
    rows = [a00, a01, a02, a10, a11, a12, a20, a21, a22,
            bx, by, bz, wx, wy, wz, zero, s2, rinv2,
            zero, zero, zero, zero, zero, zero]
    for i, v in enumerate(rows):
        tbl_ref[i:i + 1, :] = v


def _main_body(pts_ref, sp_ref, aux_ref, g_ref, nb_ref):
    Pt = pts_ref[...]
    sp = sp_ref[...]
    s2 = aux_ref[:, 0:1]
    rinv2 = aux_ref[:, 1:2]

    p2 = jnp.sum(Pt * Pt, axis=0, keepdims=True)
    sps = jax.lax.dot_general(sp, Pt, (((1,), (0,)), ((), ())),
                              preferred_element_type=jnp.float32)
    d2 = jnp.maximum(s2 + p2 - 2.0 * sps, 0.0)
    q = (d2 + 1e-12) * rinv2

    iota = jax.lax.broadcasted_iota(jnp.int32, q.shape, 0).astype(jnp.float32)
    BIG = jnp.float32(3.0e38)
    FBIG = jnp.float32(1e9)
    es = []
    q0 = None
    Z = None
    for k in range(KNN):
        m = jnp.min(q, axis=0, keepdims=True)
        idxf = jnp.min(jnp.where(q == m, iota, FBIG), axis=0,
                       keepdims=True)
        if k == 0:
            q0 = m
            e = jnp.ones_like(m)
            Z = e
        else:
            e = jnp.exp(2.0 * (q0 - m))
            Z = Z + e
        onehot = iota == idxf
        q = jnp.where(onehot, BIG, q)
        nb_ref[k:k + 1, :] = idxf.astype(jnp.int32)
        es.append(e)
    zinv = 1.0 / Z
    for k in range(KNN):
        g_ref[k:k + 1, :] = es[k] * zinv


def _sc_combine(pts_hbm, nb_hbm, g_hbm, tbl_hbm, oxyz_hbm, odr_hbm,
                pts_v, nb_v, g_v, tbl_v, ox_v, or_v, sem):
    c = lax.axis_index("c")
    s = lax.axis_index("s")
    wid = s * 2 + c
    base = wid * CHUNK
    copies = []
    for p in range(3):
        copies.append(pltpu.async_copy(
            pts_hbm.at[pl.ds(p * NPAD + base, CHUNK)],
            pts_v.at[pl.ds(p * CHUNK, CHUNK)], sem))
    for k in range(KNN):
        copies.append(pltpu.async_copy(
            nb_hbm.at[pl.ds(k * NPAD + base, CHUNK)],
            nb_v.at[pl.ds(k * CHUNK, CHUNK)], sem))
        copies.append(pltpu.async_copy(
            g_hbm.at[pl.ds(k * NPAD + base, CHUNK)],
            g_v.at[pl.ds(k * CHUNK, CHUNK)], sem))
    copies.append(pltpu.async_copy(tbl_hbm, tbl_v, sem))
    for cp in copies:
        cp.wait()

    @functools.partial(plsc.parallel_loop, 0, NGRP, unroll=2)
    def group(gi):
        b = gi * 16
        accs = [jnp.zeros((16,), jnp.float32) for _ in range(15)]
        for k in range(KNN):
            nk = nb_v[pl.ds(k * CHUNK + b, 16)]
            gk = g_v[pl.ds(k * CHUNK + b, 16)]
            row = nk * 16
            for j in range(15):
                accs[j] = accs[j] + gk * plsc.load_gather(tbl_v, [row + j])
        px = pts_v[pl.ds(b, 16)]
        py = pts_v[pl.ds(CHUNK + b, 16)]
        pz = pts_v[pl.ds(2 * CHUNK + b, 16)]
        ox_v[pl.ds(b, 16)] = accs[0] * px + accs[1] * py + accs[2] * pz + accs[9]
        ox_v[pl.ds(CHUNK + b, 16)] = accs[3] * px + accs[4] * py + accs[5] * pz + accs[10]
        ox_v[pl.ds(2 * CHUNK + b, 16)] = accs[6] * px + accs[7] * py + accs[8] * pz + accs[11]
        or_v[pl.ds(b, 16)] = accs[12]
        or_v[pl.ds(CHUNK + b, 16)] = accs[13]
        or_v[pl.ds(2 * CHUNK + b, 16)] = accs[14]

    for p in range(3):
        pltpu.sync_copy(ox_v.at[pl.ds(p * CHUNK, CHUNK)],
                        oxyz_hbm.at[pl.ds(p * NPAD + base, CHUNK)])
        pltpu.sync_copy(or_v.at[pl.ds(p * CHUNK, CHUNK)],
                        odr_hbm.at[pl.ds(p * NPAD + base, CHUNK)])


_sc_combine_call = functools.partial(
    pl.kernel,
    mesh=plsc.VectorSubcoreMesh(core_axis_name="c", subcore_axis_name="s"),
    compiler_params=pltpu.CompilerParams(needs_layout_passes=False),
    out_type=[
        jax.ShapeDtypeStruct((3 * NPAD,), jnp.float32),
        jax.ShapeDtypeStruct((3 * NPAD,), jnp.float32),
    ],
    scratch_types=[
        pltpu.VMEM((3 * CHUNK,), jnp.float32),
        pltpu.VMEM((KNN * CHUNK,), jnp.int32),
        pltpu.VMEM((KNN * CHUNK,), jnp.float32),
        pltpu.VMEM((N_SP * 16,), jnp.float32),
        pltpu.VMEM((3 * CHUNK,), jnp.float32),
        pltpu.VMEM((3 * CHUNK,), jnp.float32),
        pltpu.SemaphoreType.DMA,
    ],
)(_sc_combine)


@jax.jit
def kernel(points, sp_delta_t, sp_delta_r, sp_xyz, radius):
    n = points.shape[0]
    spt = sp_xyz.T
    sdr = sp_delta_r.T
    sdt = sp_delta_t.T
    rad2 = radius.reshape(1, N_SP)

    tbl = pl.pallas_call(
        _table_body,
        out_shape=jax.ShapeDtypeStruct((24, N_SP), jnp.float32),
    )(spt, sdr, sdt, rad2)

    pts_t = points.T
    aux = tbl[16:18, :].T

    nblocks = (n + ROWS - 1) // ROWS
    g_t, nb_t = pl.pallas_call(
        _main_body,
        grid=(nblocks,),
        in_specs=[
            pl.BlockSpec((3, ROWS), lambda i: (0, i)),
            pl.BlockSpec((N_SP, 3), lambda i: (0, 0)),
            pl.BlockSpec((N_SP, 2), lambda i: (0, 0)),
        ],
        out_specs=[
            pl.BlockSpec((KNN, ROWS), lambda i: (0, i)),
            pl.BlockSpec((KNN, ROWS), lambda i: (0, i)),
        ],
        out_shape=[
            jax.ShapeDtypeStruct((KNN, n), jnp.float32),
            jax.ShapeDtypeStruct((KNN, n), jnp.int32),
        ],
        compiler_params=pltpu.CompilerParams(
            dimension_semantics=("arbitrary",),
        ),
    )(pts_t, sp_xyz, aux)

    pad = NPAD - n
    pts_flat = jnp.pad(pts_t, ((0, 0), (0, pad))).reshape(-1)
    nb_flat = jnp.pad(nb_t, ((0, 0), (0, pad))).reshape(-1)
    g_flat = jnp.pad(g_t, ((0, 0), (0, pad))).reshape(-1)
    tbl_rows = tbl[0:16, :].T.reshape(-1)

    oxyz_t, odr_t = _sc_combine_call(pts_flat, nb_flat, g_flat, tbl_rows)
    dxyz = oxyz_t.reshape(3, NPAD)[:, :n].T
    dr = odr_t.reshape(3, NPAD)[:, :n].T
    return (dxyz, dr, g_t.T, nb_t.T)
